# baseline (speedup 1.0000x reference)
import functools

import jax
import jax.numpy as jnp
from jax import lax
from jax.experimental import pallas as pl
from jax.experimental.pallas import tpu as pltpu
from jax.experimental.pallas import tpu_sc as plsc

SCALE = 64.0
MARGIN = 0.4

CHUNK = 128
NW = 32


def _gather_rows(idx2d, table, b_pad, d):
    n_chunks = b_pad // CHUNK
    chunks_w = n_chunks // NW

    mesh = plsc.VectorSubcoreMesh(core_axis_name="c", subcore_axis_name="s")

    @functools.partial(
        pl.kernel,
        mesh=mesh,
        compiler_params=pltpu.CompilerParams(use_tc_tiling_on_sc=False),
        out_type=jax.ShapeDtypeStruct((b_pad, d), jnp.float32),
        scratch_types=[
            pltpu.VMEM((chunks_w, CHUNK), jnp.int32),
            pltpu.VMEM((2, CHUNK, d), jnp.float32),
            pltpu.SemaphoreType.DMA,
            pltpu.SemaphoreType.DMA,
        ],
    )
    def gather_kernel(idx_hbm, table_hbm, out_hbm, idx_v, rows_v, sem0, sem1):
        wid = lax.axis_index("s") * 2 + lax.axis_index("c")
        chunk0 = wid * chunks_w
        pltpu.sync_copy(idx_hbm.at[wid], idx_v)

        def start_dyn(c, buf):
            @pl.when(buf == 0)
            def _():
                pltpu.async_copy(table_hbm.at[idx_v.at[c]], rows_v.at[0], sem0)

            @pl.when(buf == 1)
            def _():
                pltpu.async_copy(table_hbm.at[idx_v.at[c]], rows_v.at[1], sem1)

        def wait_dyn(c, buf):
            @pl.when(buf == 0)
            def _():
                pltpu.make_async_copy(
                    table_hbm.at[idx_v.at[c]], rows_v.at[0], sem0).wait()

            @pl.when(buf == 1)
            def _():
                pltpu.make_async_copy(
                    table_hbm.at[idx_v.at[c]], rows_v.at[1], sem1).wait()

        def body(c, _):
            buf = lax.rem(c, 2)

            @pl.when(c + 1 < chunks_w)
            def _():
                start_dyn(c + 1, lax.rem(c + 1, 2))

            wait_dyn(c, buf)
            pltpu.sync_copy(
                rows_v.at[buf],
                out_hbm.at[pl.ds((chunk0 + c) * CHUNK, CHUNK)])
            return 0

        pltpu.async_copy(table_hbm.at[idx_v.at[0]], rows_v.at[0], sem0)
        lax.fori_loop(0, chunks_w, body, 0)

    return gather_kernel(idx2d, table)


def _fc_block(feat_ref, w_ref, lbl_ref, out_ref, *, bc):
    j = pl.program_id(0)
    w = w_ref[...]
    ss = jnp.sum(w * w, axis=1, keepdims=True)
    norm = jnp.maximum(jnp.sqrt(ss), 1e-12)
    wn = w / norm
    x = feat_ref[...]
    logits = lax.dot_general(
        x, wn, (((1,), (1,)), ((), ())),
        preferred_element_type=jnp.float32)
    lbl = lbl_ref[...]
    cols = j * bc + lax.broadcasted_iota(jnp.int32, logits.shape, 1)
    mh = jnp.where(cols == lbl, jnp.float32(MARGIN), jnp.float32(0.0))
    out_ref[...] = (logits - mh) * SCALE


def _fused_fc(features, sub_w, label2d, n_out, bc):
    b, d = features.shape
    grid = (n_out + bc - 1) // bc
    return pl.pallas_call(
        functools.partial(_fc_block, bc=bc),
        grid=(grid,),
        in_specs=[
            pl.BlockSpec((b, d), lambda j: (0, 0)),
            pl.BlockSpec((bc, d), lambda j: (j, 0)),
            pl.BlockSpec((b, 1), lambda j: (0, 0)),
        ],
        out_specs=pl.BlockSpec((b, bc), lambda j: (0, j)),
        out_shape=jax.ShapeDtypeStruct((b, n_out), jnp.float32),
    )(features, sub_w, label2d)


def kernel(total_features, weight, index, label):
    b, d = total_features.shape
    n = index.shape[0]

    pad_unit = NW * CHUNK
    b_pad = ((n + pad_unit - 1) // pad_unit) * pad_unit

    idx = index.astype(jnp.int32)
    idx_pad = jnp.concatenate(
        [idx, jnp.zeros((b_pad - n,), jnp.int32)])
    idx2d = idx_pad.reshape(NW, b_pad // (NW * CHUNK), CHUNK)

    sub_w = _gather_rows(idx2d, weight, b_pad, d)

    label2d = label.astype(jnp.int32).reshape(b, 1)
    return _fused_fc(total_features, sub_w, label2d, n, bc=2048)


# speedup vs baseline: 1.3952x; 1.3952x over previous
import functools

import jax
import jax.numpy as jnp
from jax import lax
from jax.experimental import pallas as pl
from jax.experimental.pallas import tpu as pltpu
from jax.experimental.pallas import tpu_sc as plsc

SCALE = 64.0
MARGIN = 0.4

CHUNK = 128
NW = 32


def _gather_rows(idx2d, table, b_pad, d):
    n_chunks = b_pad // CHUNK
    chunks_w = n_chunks // NW

    mesh = plsc.VectorSubcoreMesh(core_axis_name="c", subcore_axis_name="s")

    @functools.partial(
        pl.kernel,
        mesh=mesh,
        compiler_params=pltpu.CompilerParams(use_tc_tiling_on_sc=False),
        out_type=jax.ShapeDtypeStruct((b_pad, d), jnp.float32),
        scratch_types=[
            pltpu.VMEM((chunks_w, CHUNK), jnp.int32),
            pltpu.VMEM((2, CHUNK, d), jnp.float32),
            pltpu.SemaphoreType.DMA,
            pltpu.SemaphoreType.DMA,
        ],
    )
    def gather_kernel(idx_hbm, table_hbm, out_hbm, idx_v, rows_v, sem0, sem1):
        wid = lax.axis_index("s") * 2 + lax.axis_index("c")
        chunk0 = wid * chunks_w
        pltpu.sync_copy(idx_hbm.at[wid], idx_v)

        def start_dyn(c, buf):
            @pl.when(buf == 0)
            def _():
                pltpu.async_copy(table_hbm.at[idx_v.at[c]], rows_v.at[0], sem0)

            @pl.when(buf == 1)
            def _():
                pltpu.async_copy(table_hbm.at[idx_v.at[c]], rows_v.at[1], sem1)

        def wait_dyn(c, buf):
            @pl.when(buf == 0)
            def _():
                pltpu.make_async_copy(
                    table_hbm.at[idx_v.at[c]], rows_v.at[0], sem0).wait()

            @pl.when(buf == 1)
            def _():
                pltpu.make_async_copy(
                    table_hbm.at[idx_v.at[c]], rows_v.at[1], sem1).wait()

        def body(c, _):
            buf = lax.rem(c, 2)

            @pl.when(c + 1 < chunks_w)
            def _():
                start_dyn(c + 1, lax.rem(c + 1, 2))

            wait_dyn(c, buf)
            pltpu.sync_copy(
                rows_v.at[buf],
                out_hbm.at[pl.ds((chunk0 + c) * CHUNK, CHUNK)])
            return 0

        pltpu.async_copy(table_hbm.at[idx_v.at[0]], rows_v.at[0], sem0)
        lax.fori_loop(0, chunks_w, body, 0)

    return gather_kernel(idx2d, table)


def _fc_block(feat_ref, w_ref, lbl_ref, out_ref, *, bc):
    j = pl.program_id(0)
    w = w_ref[...]
    ss = jnp.sum(w * w, axis=1, keepdims=True)
    norm = jnp.maximum(jnp.sqrt(ss), 1e-12)
    wn = w / norm
    x = feat_ref[...]
    logits = lax.dot_general(
        wn, x, (((1,), (1,)), ((), ())),
        preferred_element_type=jnp.float32)
    lbl = lbl_ref[...]
    rows = j * bc + lax.broadcasted_iota(jnp.int32, logits.shape, 0)
    mh = jnp.where(rows == lbl, jnp.float32(MARGIN), jnp.float32(0.0))
    out_ref[...] = (logits - mh) * SCALE


def _fused_fc(features, sub_w, label2d, n_out, bc):
    b, d = features.shape
    grid = (n_out + bc - 1) // bc
    return pl.pallas_call(
        functools.partial(_fc_block, bc=bc),
        grid=(grid,),
        in_specs=[
            pl.BlockSpec((b, d), lambda j: (0, 0)),
            pl.BlockSpec((bc, d), lambda j: (j, 0)),
            pl.BlockSpec((1, b), lambda j: (0, 0)),
        ],
        out_specs=pl.BlockSpec((bc, b), lambda j: (j, 0)),
        out_shape=jax.ShapeDtypeStruct((n_out, b), jnp.float32),
    )(features, sub_w, label2d)


def kernel(total_features, weight, index, label):
    b, d = total_features.shape
    n = index.shape[0]

    pad_unit = NW * CHUNK
    b_pad = ((n + pad_unit - 1) // pad_unit) * pad_unit

    idx = index.astype(jnp.int32)
    idx_pad = jnp.concatenate(
        [idx, jnp.zeros((b_pad - n,), jnp.int32)])
    idx2d = idx_pad.reshape(NW, b_pad // (NW * CHUNK), CHUNK)

    sub_w = _gather_rows(idx2d, weight, b_pad, d)

    label2d = label.astype(jnp.int32).reshape(1, b)
    out_t = _fused_fc(total_features, sub_w, label2d, n, bc=2048)
    return out_t.T
